# baseline (speedup 1.0000x reference)
import numpy as np

import jax
import jax.numpy as jnp
from jax import lax
from jax.experimental import pallas as pl
from jax.experimental.pallas import tpu as pltpu

F32 = jnp.float32
BF16 = jnp.bfloat16
NB = 1024

_E1 = np.zeros((25, 4 * 258), np.float32)
_E2 = np.zeros((150, 4 * 780), np.float32)
for _dy in range(2):
    for _dx in range(2):
        _off = 2 * _dy + _dx
        for _i in range(5):
            for _j in range(5):
                _tap = 5 * _i + _j
                _E1[_tap, 258 * _off + 32 * (_dy + _i) + (_dx + _j)] = 1.0
                for _ci in range(6):
                    _E2[6 * _tap + _ci,
                        780 * _off + 6 * (16 * (_dy + _i) + (_dx + _j))
                        + _ci] = 1.0


def _lenet_tile_kernel(x_ref, b1a_ref, b1z_ref, b2c_ref, w1m_ref, w2p_ref,
                       w3p_ref, bp1_ref, bp2_ref, bf1_ref, bf2_ref, bf3_ref,
                       o_ref, p1_s, z_s):
    for u in range(14):
        if u < 13:
            out = jnp.dot(b1a_ref[...], x_ref[64 * u:64 * u + 256, :],
                          preferred_element_type=F32)
        else:
            out = jnp.dot(b1z_ref[...], x_ref[832:1024, :],
                          preferred_element_type=F32)
        o4 = out.reshape(4, 96, NB)
        mx = jnp.maximum(jnp.maximum(o4[0], o4[1]),
                         jnp.maximum(o4[2], o4[3]))
        mx = jnp.maximum(mx + bp1_ref[...], 0.0)
        p1_s[96 * u:96 * u + 96, :] = mx.astype(BF16)
    p1_s[1344:1536, :] = jnp.zeros((192, NB), BF16)

    for s in range(5):
        y = jnp.dot(b2c_ref[...], p1_s[192 * s:192 * s + 768, :],
                    preferred_element_type=F32)
        y4 = y.reshape(4, 80, NB)
        my = jnp.maximum(jnp.maximum(y4[0], y4[1]),
                         jnp.maximum(y4[2], y4[3]))
        my = jnp.maximum(my + bp2_ref[...], 0.0)
        z_s[80 * s:80 * s + 80, :] = my.astype(BF16)

    h = jnp.dot(w1m_ref[...], z_s[...], preferred_element_type=F32)
    h = jnp.maximum(h + bf1_ref[...], 0.0).astype(BF16)
    h = lax.dot_general(w2p_ref[...], h, (((0,), (0,)), ((), ())),
                        preferred_element_type=F32)
    h = jnp.maximum(h + bf2_ref[...], 0.0).astype(BF16)
    o = lax.dot_general(w3p_ref[...], h, (((0,), (0,)), ((), ())),
                        preferred_element_type=F32)
    o_ref[...] = o[:16, :] + bf3_ref[...]


def kernel(x, w1p, b1p, w2p, b2p, fc1p, bfc1, fc2p, bfc2, fc3p, bfc3):
    n = x.shape[0]
    nt = n // NB

    xt = x.reshape(n, 1024).astype(BF16).T

    pat1 = lax.dot_general(w1p[:, :6], jnp.asarray(_E1),
                           (((0,), (0,)), ((), ())))
    pat1 = jnp.transpose(pat1.reshape(6, 4, 258), (1, 0, 2))
    b1a = jnp.broadcast_to(pat1.reshape(4, 6, 1, 258),
                           (4, 6, 16, 258)).reshape(4, 6, 16 * 258)
    b1a = b1a[:, :, :16 * 256].reshape(4, 6, 16, 256)
    b1a = jnp.transpose(b1a, (0, 2, 1, 3)).reshape(384, 256).astype(BF16)
    b1z = b1a[:, :192]

    pat2 = lax.dot_general(w2p[:, :6, :16].reshape(150, 16), jnp.asarray(_E2),
                           (((0,), (0,)), ((), ())))
    pat2 = jnp.transpose(pat2.reshape(16, 4, 780), (1, 0, 2))
    b2c = jnp.broadcast_to(pat2.reshape(4, 16, 1, 780),
                           (4, 16, 5, 780)).reshape(4, 16, 5 * 780)
    b2c = b2c[:, :, :5 * 768].reshape(320, 768).astype(BF16)

    w1m = jnp.transpose(fc1p[:, :16, :120], (2, 1, 0)).reshape(120, 16, 5, 5)
    w1m = jnp.transpose(w1m, (0, 2, 1, 3)).reshape(120, 400)
    w1m = jnp.pad(w1m, ((0, 8), (0, 0))).astype(BF16)

    bp1 = jnp.broadcast_to(b1p[0, :6][None, :, None], (16, 6, NB)).reshape(
        96, NB)
    bp2 = jnp.broadcast_to(b2p[0, :16][:, None, None], (16, 5, NB)).reshape(
        80, NB)
    bf1 = jnp.broadcast_to(bfc1.T, (128, NB))
    bf2 = jnp.broadcast_to(bfc2.T, (128, NB))
    bf3 = jnp.broadcast_to(bfc3.T[:16], (16, NB))

    out = pl.pallas_call(
        _lenet_tile_kernel,
        out_shape=jax.ShapeDtypeStruct((16, n), F32),
        grid=(nt,),
        in_specs=[
            pl.BlockSpec((1024, NB), lambda b: (0, b)),
            pl.BlockSpec((384, 256), lambda b: (0, 0)),
            pl.BlockSpec((384, 192), lambda b: (0, 0)),
            pl.BlockSpec((320, 768), lambda b: (0, 0)),
            pl.BlockSpec((128, 400), lambda b: (0, 0)),
            pl.BlockSpec((128, 128), lambda b: (0, 0)),
            pl.BlockSpec((128, 128), lambda b: (0, 0)),
            pl.BlockSpec((96, NB), lambda b: (0, 0)),
            pl.BlockSpec((80, NB), lambda b: (0, 0)),
            pl.BlockSpec((128, NB), lambda b: (0, 0)),
            pl.BlockSpec((128, NB), lambda b: (0, 0)),
            pl.BlockSpec((16, NB), lambda b: (0, 0)),
        ],
        out_specs=pl.BlockSpec((16, NB), lambda b: (0, b)),
        scratch_shapes=[
            pltpu.VMEM((1536, NB), BF16),
            pltpu.VMEM((400, NB), BF16),
        ],
        compiler_params=pltpu.CompilerParams(
            dimension_semantics=("parallel",),
            vmem_limit_bytes=48 * 1024 * 1024,
        ),
    )(xt, b1a, b1z, b2c, w1m, fc2p.astype(BF16), fc3p.astype(BF16),
      bp1, bp2, bf1, bf2, bf3)

    return out[:10, :].T
